# baseline (speedup 1.0000x reference)
import functools

import jax
import jax.numpy as jnp
import numpy as np
from jax.experimental import pallas as pl
from jax.experimental.pallas import tpu as pltpu

N_NODES = 50000
NUM_GRAPHS = 50
HEADS = 4
HID = 16
NCHIP = 18
NEDGE = 3
RATIO = 0.5

_PB = 64
_PF = 128


def _pad2(a, r, c):
    return jnp.zeros((r, c), jnp.float32).at[: a.shape[0], : a.shape[1]].set(a)


def _head_kernel(h_ref, p_ref, *refs):
    out_ref = refs[-1]
    fcW = refs[0:5]
    fcb = refs[5:10]
    pmW = refs[10:15]
    pmb = refs[15:20]
    Wh = refs[20]
    Wp = refs[21]
    h = h_ref[:]
    for i in range(5):
        h = jax.nn.relu(
            jnp.dot(h, fcW[i][:], preferred_element_type=jnp.float32)
            + fcb[i][0:1, :]
        )
    p = p_ref[:]
    for i in range(5):
        p = jax.nn.relu(
            jnp.dot(p, pmW[i][:], preferred_element_type=jnp.float32)
            + pmb[i][0:1, :]
        )
    out_ref[:] = (
        jnp.dot(h, Wh[:], preferred_element_type=jnp.float32)
        + jnp.dot(p, Wp[:], preferred_element_type=jnp.float32)
    )


def _mlp_head(xg, prom_x, params):
    h = _pad2(xg, _PB, _PF)
    p = _pad2(prom_x, _PB, _PF)
    ops = [h, p]
    for l in params["fc"]:
        ops.append(_pad2(l["W"], _PF, _PF))
    for l in params["fc"]:
        ops.append(_pad2(l["b"][None, :], 8, _PF))
    for l in params["prom"]:
        ops.append(_pad2(l["W"], _PF, _PF))
    for l in params["prom"]:
        ops.append(_pad2(l["b"][None, :], 8, _PF))
    rW = params["readout_W"]
    ops.append(_pad2(rW[0:2], _PF, _PF))
    ops.append(_pad2(rW[2:4], _PF, _PF))
    out = pl.pallas_call(
        _head_kernel,
        out_shape=jax.ShapeDtypeStruct((_PB, _PF), jnp.float32),
    )(*ops)
    return out[:NUM_GRAPHS, 0:1] + params["readout_b"]


def _wegat_conv(x, edge_attr, src, dst, edge_mask, cp):
    n = x.shape[0]
    xh = (x @ cp["Wx"]).reshape(n, HEADS, HID)
    eh = (edge_attr @ cp["We"]).reshape(-1, HEADS, NEDGE)
    att = cp["att"]
    att_i = att[:, :HID]
    att_j = att[:, HID:2 * HID]
    att_e = att[:, 2 * HID:]
    a_i = (xh * att_i[None, :, :]).sum(-1)
    a_j = (xh * att_j[None, :, :]).sum(-1)
    a_e = (eh * att_e[None, :, :]).sum(-1)
    alpha = jax.nn.leaky_relu(a_i[dst] + a_j[src] + a_e, 0.2)
    alpha = jnp.where(edge_mask[:, None] > 0, alpha, -1e9)
    amax = jax.ops.segment_max(alpha, dst, num_segments=n)
    amax = jnp.maximum(amax, -1e9)
    ex = jnp.exp(alpha - amax[dst]) * edge_mask[:, None]
    denom = jax.ops.segment_sum(ex, dst, num_segments=n) + 1e-16
    a = ex / denom[dst]
    msg = a[:, :, None] * xh[src]
    out = jax.ops.segment_sum(msg, dst, num_segments=n).mean(axis=1)
    e_new = eh.mean(axis=1)
    return out, e_new


def _topk_pool(x, node_mask, batch, w):
    n = x.shape[0]
    score = jnp.tanh((x @ w) / (jnp.linalg.norm(w) + 1e-16))
    score_m = jnp.where(node_mask > 0, score, -2.0)
    comp = batch.astype(jnp.float32) * 10.0 - score_m
    order = jnp.argsort(comp)
    bs = batch[order]
    pos = jnp.arange(n)
    gstart = jax.ops.segment_min(pos, bs, num_segments=NUM_GRAPHS)
    rank_sorted = pos - gstart[bs]
    rank = jnp.zeros((n,), jnp.int32).at[order].set(rank_sorted.astype(jnp.int32))
    n_active = jax.ops.segment_sum(node_mask, batch, num_segments=NUM_GRAPHS)
    k = jnp.ceil(RATIO * n_active)
    keep = (rank.astype(jnp.float32) < k[batch]) & (node_mask > 0)
    keep_f = keep.astype(jnp.float32)
    x = x * score[:, None] * keep_f[:, None]
    return x, keep_f


def kernel(x, edge_attr, edge_index, prom_x, batch, params):
    x = jnp.where(jnp.isnan(x), 0.0, x).astype(jnp.float32)
    edge_attr = jnp.where(jnp.isnan(edge_attr), 0.0, edge_attr).astype(jnp.float32)
    prom_x = jnp.reshape(prom_x, (-1, NCHIP)).astype(jnp.float32)
    prom_x = jnp.where(jnp.isnan(prom_x), 0.0, prom_x)
    src = edge_index[0]
    dst = edge_index[1]
    node_mask = jnp.ones((x.shape[0],), jnp.float32)
    edge_mask = jnp.ones((src.shape[0],), jnp.float32)
    for cp in params["convs"]:
        x, edge_attr = _wegat_conv(x, edge_attr, src, dst, edge_mask, cp)
        x = jax.nn.relu(x)
        edge_attr = jax.nn.relu(edge_attr)
        x, node_mask = _topk_pool(x, node_mask, batch, cp["pool_w"])
        edge_mask = edge_mask * node_mask[src] * node_mask[dst]
    xg = jax.ops.segment_max(
        jnp.where(node_mask[:, None] > 0, x, -1e9), batch, num_segments=NUM_GRAPHS
    )
    xg = jnp.where(jnp.isfinite(xg), xg, 0.0)
    return _mlp_head(xg, prom_x, params)


# speedup vs baseline: 61.6115x; 61.6115x over previous
import functools

import jax
import jax.numpy as jnp
import numpy as np
from jax import lax
from jax.experimental import pallas as pl
from jax.experimental.pallas import tpu as pltpu
from jax.experimental.pallas import tpu_sc as plsc

N_NODES = 50000
N_EDGES = 800000
NUM_GRAPHS = 50
HEADS = 4
HID = 16
NCHIP = 18
NEDGE = 3
RATIO = 0.5

N_PAD = 50176
E_PAD = 819200
CHUNK = 512
NT = 16
NCH = E_PAD // (NT * CHUNK)
ROWS_T = N_PAD // NT
DR = 112

_PB = 64
_PF = 128


def _edge_body(src_hbm, dst_hbm, t_hbm, xh2_hbm, ae_hbm, z16_hbm, z1_hbm,
               out_hbm,
               sp, dp, aib, ajb, aev, exb, xs, mbuf, dbuf, obuf,
               tai, taj, xsp, acc_msg, acc_den, sem):
    c = lax.axis_index("c")
    tid = lax.axis_index("s")
    r0 = tid * ROWS_T

    nsl = pl.ds(r0, ROWS_T)
    pltpu.sync_copy(t_hbm.at[c, 0, nsl], tai.at[nsl])
    pltpu.sync_copy(t_hbm.at[c, 1, nsl], taj.at[nsl])
    pltpu.sync_copy(xh2_hbm.at[c, nsl], xsp.at[nsl])
    pltpu.sync_copy(z16_hbm.at[nsl], acc_msg.at[nsl])
    pltpu.sync_copy(z1_hbm.at[nsl], acc_den.at[nsl])
    plsc.subcore_barrier()

    def _chunk(i, _):
        base = (tid * NCH + i) * CHUNK
        row4 = (tid * NCH + i) * (CHUNK // 128)
        pltpu.sync_copy(src_hbm.at[pl.ds(row4, CHUNK // 128)], sp)
        pltpu.sync_copy(dst_hbm.at[pl.ds(row4, CHUNK // 128)], dp)

        descs = []
        for j in range(CHUNK // 128):
            dsl = pl.ds(j * 128, 128)
            descs.append(pltpu.async_copy(tai.at[dp.at[j]], aib.at[dsl], sem))
            descs.append(pltpu.async_copy(taj.at[sp.at[j]], ajb.at[dsl], sem))
            descs.append(pltpu.async_copy(xsp.at[sp.at[j]], xs.at[dsl], sem))
        pltpu.sync_copy(ae_hbm.at[c, pl.ds(base, CHUNK)], aev)
        for d in descs:
            d.wait()

        def _alpha(j, _):
            sl = pl.ds(j * 16, 16)
            z = aib[sl] + ajb[sl] + aev[sl]
            e = jnp.exp(jnp.maximum(z, 0.2 * z))
            exb[sl] = e
            for k in range(16):
                ei = j * 16 + k
                xs[ei, pl.ds(0, 16)] = xs[ei, pl.ds(0, 16)] * e[k]
            return 0
        lax.fori_loop(0, CHUNK // 16, _alpha, 0)

        for j in range(CHUNK // 128):
            dsl = pl.ds(j * 128, 128)
            pltpu.sync_copy(xs.at[dsl], acc_msg.at[dp.at[j]], add=True)
            pltpu.sync_copy(exb.at[dsl], acc_den.at[dp.at[j]], add=True)
        return 0
    lax.fori_loop(0, NCH, _chunk, 0)
    plsc.subcore_barrier()

    def _drain(q, _):
        rq = r0 + q * DR
        pltpu.sync_copy(acc_msg.at[pl.ds(rq, DR)], mbuf)
        pltpu.sync_copy(acc_den.at[pl.ds(rq, DR)], dbuf)

        def _node(j, _):
            rec = 1.0 / (dbuf[pl.ds(j * 16, 16)] + 1e-16)
            for k in range(16):
                i = j * 16 + k
                obuf[i, pl.ds(0, 16)] = mbuf[i, pl.ds(0, 16)] * rec[k]
            return 0
        lax.fori_loop(0, DR // 16, _node, 0)
        pltpu.sync_copy(obuf, out_hbm.at[c, pl.ds(rq, DR)])
        return 0
    lax.fori_loop(0, ROWS_T // DR, _drain, 0)


@functools.lru_cache(maxsize=1)
def _edge_call():
    mesh = plsc.VectorSubcoreMesh(core_axis_name="c", subcore_axis_name="s")
    ev = lambda: pltpu.VMEM((CHUNK,), jnp.float32)
    return pl.kernel(
        _edge_body,
        out_type=jax.ShapeDtypeStruct((2, N_PAD, HID), jnp.float32),
        mesh=mesh,
        compiler_params=pltpu.CompilerParams(use_tc_tiling_on_sc=False),
        scratch_types=[
            pltpu.VMEM((CHUNK // 128, 128), jnp.int32),
            pltpu.VMEM((CHUNK // 128, 128), jnp.int32),
            ev(), ev(), ev(), ev(),
            pltpu.VMEM((CHUNK, HID), jnp.float32),
            pltpu.VMEM((DR, HID), jnp.float32),
            pltpu.VMEM((DR,), jnp.float32),
            pltpu.VMEM((DR, HID), jnp.float32),
            pltpu.VMEM_SHARED((N_PAD,), jnp.float32),
            pltpu.VMEM_SHARED((N_PAD,), jnp.float32),
            pltpu.VMEM_SHARED((N_PAD, HID), jnp.float32),
            pltpu.VMEM_SHARED((N_PAD, HID), jnp.float32),
            pltpu.VMEM_SHARED((N_PAD,), jnp.float32),
            pltpu.SemaphoreType.DMA,
        ],
    )


def _conv_edge_sc(x, edge_attr, src_p, dst_p, node_mask, cp):
    n = x.shape[0]
    xh = x @ cp["Wx"]
    xh4 = xh.reshape(n, HEADS, HID)
    att = cp["att"]
    att_i = att[:, :HID]
    att_j = att[:, HID:2 * HID]
    att_e = att[:, 2 * HID:]
    ai = jnp.einsum("nhd,hd->nh", xh4, att_i)
    aj = jnp.einsum("nhd,hd->nh", xh4, att_j)
    moff = jnp.where(node_mask > 0, 0.0, -1e9)
    ai_m = (ai + moff[:, None]).T
    aj_m = (aj + moff[:, None]).T
    ai_m = jnp.pad(ai_m, ((0, 0), (0, N_PAD - n)))
    aj_m = jnp.pad(aj_m, ((0, 0), (0, N_PAD - n)))
    xh4p = jnp.pad(xh4, ((0, N_PAD - n), (0, 0), (0, 0)))
    Me = (cp["We"].reshape(-1, HEADS, NEDGE) * att_e[None]).sum(-1)
    ae = edge_attr @ Me
    ae = jnp.pad(ae, ((0, E_PAD - ae.shape[0]), (0, 0)),
                 constant_values=-1e9).T
    z16 = jnp.zeros((N_PAD, HID), jnp.float32)
    z1 = jnp.zeros((N_PAD,), jnp.float32)
    call = _edge_call()
    acc = None
    for t in range(2):
        t_hbm = jnp.stack([jnp.stack([ai_m[2 * t + c], aj_m[2 * t + c]])
                           for c in range(2)])
        xh2 = jnp.stack([xh4p[:, 2 * t + 0], xh4p[:, 2 * t + 1]])
        ae2 = jnp.stack([ae[2 * t + 0], ae[2 * t + 1]])
        o = call(src_p, dst_p, t_hbm, xh2, ae2, z16, z1)
        part = o[0] + o[1]
        acc = part if acc is None else acc + part
    x_new = 0.25 * acc[:n]
    We_mean = cp["We"].reshape(-1, HEADS, NEDGE).mean(1)
    e_new = edge_attr @ We_mean
    return x_new, e_new


def _topk_pool(x, node_mask, batch, w):
    n = x.shape[0]
    score = jnp.tanh((x @ w) / (jnp.linalg.norm(w) + 1e-16))
    score_m = jnp.where(node_mask > 0, score, -2.0)
    comp = batch.astype(jnp.float32) * 10.0 - score_m
    order = jnp.argsort(comp)
    bs = batch[order]
    pos = jnp.arange(n)
    gstart = jax.ops.segment_min(pos, bs, num_segments=NUM_GRAPHS)
    rank_sorted = pos - gstart[bs]
    rank = jnp.zeros((n,), jnp.int32).at[order].set(rank_sorted.astype(jnp.int32))
    n_active = jax.ops.segment_sum(node_mask, batch, num_segments=NUM_GRAPHS)
    k = jnp.ceil(RATIO * n_active)
    keep = (rank.astype(jnp.float32) < k[batch]) & (node_mask > 0)
    keep_f = keep.astype(jnp.float32)
    x = x * score[:, None] * keep_f[:, None]
    return x, keep_f


def _pad2(a, r, c):
    return jnp.zeros((r, c), jnp.float32).at[: a.shape[0], : a.shape[1]].set(a)


def _head_kernel(h_ref, p_ref, *refs):
    out_ref = refs[-1]
    fcW = refs[0:5]
    fcb = refs[5:10]
    pmW = refs[10:15]
    pmb = refs[15:20]
    Wh = refs[20]
    Wp = refs[21]
    h = h_ref[:]
    for i in range(5):
        h = jax.nn.relu(
            jnp.dot(h, fcW[i][:], preferred_element_type=jnp.float32)
            + fcb[i][0:1, :])
    p = p_ref[:]
    for i in range(5):
        p = jax.nn.relu(
            jnp.dot(p, pmW[i][:], preferred_element_type=jnp.float32)
            + pmb[i][0:1, :])
    out_ref[:] = (jnp.dot(h, Wh[:], preferred_element_type=jnp.float32)
                  + jnp.dot(p, Wp[:], preferred_element_type=jnp.float32))


def _mlp_head(xg, prom_x, params):
    h = _pad2(xg, _PB, _PF)
    p = _pad2(prom_x, _PB, _PF)
    ops = [h, p]
    for l in params["fc"]:
        ops.append(_pad2(l["W"], _PF, _PF))
    for l in params["fc"]:
        ops.append(_pad2(l["b"][None, :], 8, _PF))
    for l in params["prom"]:
        ops.append(_pad2(l["W"], _PF, _PF))
    for l in params["prom"]:
        ops.append(_pad2(l["b"][None, :], 8, _PF))
    rW = params["readout_W"]
    ops.append(_pad2(rW[0:2], _PF, _PF))
    ops.append(_pad2(rW[2:4], _PF, _PF))
    out = pl.pallas_call(
        _head_kernel,
        out_shape=jax.ShapeDtypeStruct((_PB, _PF), jnp.float32),
    )(*ops)
    return out[:NUM_GRAPHS, 0:1] + params["readout_b"]


def kernel(x, edge_attr, edge_index, prom_x, batch, params):
    x = jnp.where(jnp.isnan(x), 0.0, x).astype(jnp.float32)
    edge_attr = jnp.where(jnp.isnan(edge_attr), 0.0, edge_attr).astype(jnp.float32)
    prom_x = jnp.reshape(prom_x, (-1, NCHIP)).astype(jnp.float32)
    prom_x = jnp.where(jnp.isnan(prom_x), 0.0, prom_x)
    src = edge_index[0]
    dst = edge_index[1]
    src_p = jnp.pad(src.astype(jnp.int32), (0, E_PAD - N_EDGES)).reshape(-1, 128)
    dst_p = jnp.pad(dst.astype(jnp.int32), (0, E_PAD - N_EDGES)).reshape(-1, 128)
    node_mask = jnp.ones((x.shape[0],), jnp.float32)
    for cp in params["convs"]:
        x, edge_attr = _conv_edge_sc(x, edge_attr, src_p, dst_p, node_mask, cp)
        x = jax.nn.relu(x)
        edge_attr = jax.nn.relu(edge_attr)
        x, node_mask = _topk_pool(x, node_mask, batch, cp["pool_w"])
    xg = jax.ops.segment_max(
        jnp.where(node_mask[:, None] > 0, x, -1e9), batch,
        num_segments=NUM_GRAPHS)
    xg = jnp.where(jnp.isfinite(xg), xg, 0.0)
    return _mlp_head(xg, prom_x, params)


# speedup vs baseline: 76.9338x; 1.2487x over previous
import functools

import jax
import jax.numpy as jnp
import numpy as np
from jax import lax
from jax.experimental import pallas as pl
from jax.experimental.pallas import tpu as pltpu
from jax.experimental.pallas import tpu_sc as plsc

N_NODES = 50000
N_EDGES = 800000
NUM_GRAPHS = 50
HEADS = 4
HID = 16
NCHIP = 18
NEDGE = 3
RATIO = 0.5

N_PAD = 50176
E_PAD = 819200
CHUNK = 512
NT = 16
NCH = E_PAD // (NT * CHUNK)
ROWS_T = N_PAD // NT
DR = 112

_PB = 64
_PF = 128


def _edge_body(src_hbm, dst_hbm, t_hbm, xh2_hbm, ae_hbm, z16_hbm, z1_hbm,
               out_hbm,
               sp, dp, aib, ajb, aev, exb, xs, mbuf, dbuf, obuf,
               tai, taj, xsp, acc_msg, acc_den, sem):
    c = lax.axis_index("c")
    tid = lax.axis_index("s")
    r0 = tid * ROWS_T

    nsl = pl.ds(r0, ROWS_T)
    pltpu.sync_copy(t_hbm.at[c, 0, nsl], tai.at[nsl])
    pltpu.sync_copy(t_hbm.at[c, 1, nsl], taj.at[nsl])
    pltpu.sync_copy(xh2_hbm.at[c, nsl], xsp.at[nsl])
    pltpu.sync_copy(z16_hbm.at[nsl], acc_msg.at[nsl])
    pltpu.sync_copy(z1_hbm.at[nsl], acc_den.at[nsl])
    plsc.subcore_barrier()

    def _chunk(i, _):
        base = (tid * NCH + i) * CHUNK
        row4 = (tid * NCH + i) * (CHUNK // 128)
        pltpu.sync_copy(src_hbm.at[pl.ds(row4, CHUNK // 128)], sp)
        pltpu.sync_copy(dst_hbm.at[pl.ds(row4, CHUNK // 128)], dp)

        descs = []
        for j in range(CHUNK // 128):
            dsl = pl.ds(j * 128, 128)
            descs.append(pltpu.async_copy(tai.at[dp.at[j]], aib.at[dsl], sem))
            descs.append(pltpu.async_copy(taj.at[sp.at[j]], ajb.at[dsl], sem))
            descs.append(pltpu.async_copy(xsp.at[sp.at[j]], xs.at[dsl], sem))
        pltpu.sync_copy(ae_hbm.at[c, pl.ds(base, CHUNK)], aev)
        for d in descs:
            d.wait()

        def _alpha(j, _):
            sl = pl.ds(j * 16, 16)
            z = aib[sl] + ajb[sl] + aev[sl]
            e = jnp.exp(jnp.maximum(z, 0.2 * z))
            exb[sl] = e
            for k in range(16):
                ei = j * 16 + k
                xs[ei, pl.ds(0, 16)] = xs[ei, pl.ds(0, 16)] * e[k]
            return 0
        lax.fori_loop(0, CHUNK // 16, _alpha, 0)

        for j in range(CHUNK // 128):
            dsl = pl.ds(j * 128, 128)
            pltpu.sync_copy(xs.at[dsl], acc_msg.at[dp.at[j]], add=True)
            pltpu.sync_copy(exb.at[dsl], acc_den.at[dp.at[j]], add=True)
        return 0
    lax.fori_loop(0, NCH, _chunk, 0)
    plsc.subcore_barrier()

    def _drain(q, _):
        rq = r0 + q * DR
        pltpu.sync_copy(acc_msg.at[pl.ds(rq, DR)], mbuf)
        pltpu.sync_copy(acc_den.at[pl.ds(rq, DR)], dbuf)

        def _node(j, _):
            rec = 1.0 / (dbuf[pl.ds(j * 16, 16)] + 1e-16)
            for k in range(16):
                i = j * 16 + k
                obuf[i, pl.ds(0, 16)] = mbuf[i, pl.ds(0, 16)] * rec[k]
            return 0
        lax.fori_loop(0, DR // 16, _node, 0)
        pltpu.sync_copy(obuf, out_hbm.at[c, pl.ds(rq, DR)])
        return 0
    lax.fori_loop(0, ROWS_T // DR, _drain, 0)


@functools.lru_cache(maxsize=1)
def _edge_call():
    mesh = plsc.VectorSubcoreMesh(core_axis_name="c", subcore_axis_name="s")
    ev = lambda: pltpu.VMEM((CHUNK,), jnp.float32)
    return pl.kernel(
        _edge_body,
        out_type=jax.ShapeDtypeStruct((2, N_PAD, HID), jnp.float32),
        mesh=mesh,
        compiler_params=pltpu.CompilerParams(use_tc_tiling_on_sc=False),
        scratch_types=[
            pltpu.VMEM((CHUNK // 128, 128), jnp.int32),
            pltpu.VMEM((CHUNK // 128, 128), jnp.int32),
            ev(), ev(), ev(), ev(),
            pltpu.VMEM((CHUNK, HID), jnp.float32),
            pltpu.VMEM((DR, HID), jnp.float32),
            pltpu.VMEM((DR,), jnp.float32),
            pltpu.VMEM((DR, HID), jnp.float32),
            pltpu.VMEM_SHARED((N_PAD,), jnp.float32),
            pltpu.VMEM_SHARED((N_PAD,), jnp.float32),
            pltpu.VMEM_SHARED((N_PAD, HID), jnp.float32),
            pltpu.VMEM_SHARED((N_PAD, HID), jnp.float32),
            pltpu.VMEM_SHARED((N_PAD,), jnp.float32),
            pltpu.SemaphoreType.DMA,
        ],
    )


def _conv_edge_sc(x, edge_attr, src_p, dst_p, node_mask, cp):
    n = x.shape[0]
    xh = x @ cp["Wx"]
    xh4 = xh.reshape(n, HEADS, HID)
    att = cp["att"]
    att_i = att[:, :HID]
    att_j = att[:, HID:2 * HID]
    att_e = att[:, 2 * HID:]
    ai = jnp.einsum("nhd,hd->nh", xh4, att_i)
    aj = jnp.einsum("nhd,hd->nh", xh4, att_j)
    moff = jnp.where(node_mask > 0, 0.0, -1e9)
    ai_m = (ai + moff[:, None]).T
    aj_m = (aj + moff[:, None]).T
    ai_m = jnp.pad(ai_m, ((0, 0), (0, N_PAD - n)))
    aj_m = jnp.pad(aj_m, ((0, 0), (0, N_PAD - n)))
    xh4p = jnp.pad(xh4, ((0, N_PAD - n), (0, 0), (0, 0)))
    Me = (cp["We"].reshape(-1, HEADS, NEDGE) * att_e[None]).sum(-1)
    ae = edge_attr @ Me
    ae = jnp.pad(ae, ((0, E_PAD - ae.shape[0]), (0, 0)),
                 constant_values=-1e9).T
    z16 = jnp.zeros((N_PAD, HID), jnp.float32)
    z1 = jnp.zeros((N_PAD,), jnp.float32)
    call = _edge_call()
    acc = None
    for t in range(2):
        t_hbm = jnp.stack([jnp.stack([ai_m[2 * t + c], aj_m[2 * t + c]])
                           for c in range(2)])
        xh2 = jnp.stack([xh4p[:, 2 * t + 0], xh4p[:, 2 * t + 1]])
        ae2 = jnp.stack([ae[2 * t + 0], ae[2 * t + 1]])
        o = call(src_p, dst_p, t_hbm, xh2, ae2, z16, z1)
        part = o[0] + o[1]
        acc = part if acc is None else acc + part
    x_new = 0.25 * acc[:n]
    We_mean = cp["We"].reshape(-1, HEADS, NEDGE).mean(1)
    e_new = edge_attr @ We_mean
    return x_new, e_new


SBUF = 50432
G_PAD = 64
IMIN = np.int32(-2147483648)
IMAX = np.int32(2147483647)


def _vsum16(v):
    s = v[0]
    for k in range(1, 16):
        s = s + v[k]
    return s


def _topk_body(keys_hbm, ginfo_hbm, thr_hbm, sbuf, gbuf, obuf, sem):
    c = lax.axis_index("c")
    tid = lax.axis_index("s")
    w = c * NT + tid
    iota = jnp.arange(16, dtype=jnp.int32)

    def do_graph(g):
        pltpu.sync_copy(ginfo_hbm.at[g], gbuf)
        v = gbuf[pl.ds(0, 16)]
        g0 = v[0]
        gc = v[1]
        kact = v[2]
        a0 = (g0 // 8) * 8
        shift = g0 - a0
        nst = (gc + shift + 127) // 128

        def _stage(i, _):
            pltpu.sync_copy(keys_hbm.at[pl.ds(a0 + i * 128, 128)],
                            sbuf.at[pl.ds(i * 128, 128)])
            return 0
        lax.fori_loop(0, nst, _stage, 0)

        nch = (gc + shift + 15) // 16

        def _key(i, acc):
            sl = pl.ds(i * 16, 16)
            kv = sbuf[sl]
            sidx = i * 16 + iota
            valid = (sidx >= shift) & (sidx < shift + gc)
            acc = acc + jnp.where(valid & (kv < kact), 1, 0)
            sbuf[sl] = jnp.where(valid, kv, IMAX)
            return acc
        na = _vsum16(lax.fori_loop(0, nch, _key,
                                   jnp.zeros((16,), jnp.int32)))
        k = (na + 1) // 2

        def count_le(t):
            def body(i, acc):
                kv = sbuf[pl.ds(i * 16, 16)]
                return acc + jnp.where(kv <= t, 1, 0)
            return _vsum16(lax.fori_loop(0, nch, body,
                                         jnp.zeros((16,), jnp.int32)))

        def _bs(_, carry):
            lo, hi = carry
            mid = (lo >> 1) + (hi >> 1) + (lo & hi & 1)
            pred = count_le(mid) >= k
            return (jnp.where(pred, lo, mid), jnp.where(pred, mid, hi))
        _, kstar = lax.fori_loop(0, 32, _bs, (IMIN, IMAX))

        r = k - count_le(kstar - 1)

        def count_eq_le(m):
            def body(i, acc):
                kv = sbuf[pl.ds(i * 16, 16)]
                sidx = i * 16 + iota
                return acc + jnp.where((kv == kstar) & (sidx <= m), 1, 0)
            return _vsum16(lax.fori_loop(0, nch, body,
                                         jnp.zeros((16,), jnp.int32)))

        def _bs2(_, carry):
            lo, hi = carry
            mid = (lo >> 1) + (hi >> 1) + (lo & hi & 1)
            pred = count_eq_le(mid) >= r
            return (jnp.where(pred, lo, mid), jnp.where(pred, mid, hi))
        _, m2 = lax.fori_loop(0, 17, _bs2,
                              (jnp.int32(-1), nch * 16))
        idx_thr = m2 - shift

        obuf[pl.ds(0, 16)] = jnp.full((16,), kstar, jnp.int32)
        pltpu.sync_copy(obuf, thr_hbm.at[0, g])
        obuf[pl.ds(0, 16)] = jnp.full((16,), idx_thr, jnp.int32)
        pltpu.sync_copy(obuf, thr_hbm.at[1, g])

    do_graph(w)
    g2 = w + 32

    @pl.when(g2 < NUM_GRAPHS)
    def _():
        do_graph(g2)


@functools.lru_cache(maxsize=1)
def _topk_call():
    mesh = plsc.VectorSubcoreMesh(core_axis_name="c", subcore_axis_name="s")
    return pl.kernel(
        _topk_body,
        out_type=jax.ShapeDtypeStruct((2, G_PAD, 16), jnp.int32),
        mesh=mesh,
        compiler_params=pltpu.CompilerParams(use_tc_tiling_on_sc=False),
        scratch_types=[
            pltpu.VMEM((SBUF,), jnp.int32),
            pltpu.VMEM((16,), jnp.int32),
            pltpu.VMEM((16,), jnp.int32),
            pltpu.SemaphoreType.DMA,
        ],
    )


def _f32key(v):
    b = jax.lax.bitcast_convert_type(v + 0.0, jnp.int32)
    return jnp.where(b >= 0, b, jnp.bitwise_xor(jnp.bitwise_not(b), IMIN))


def _topk_pool_sc(x, node_mask, batch, w, ginfo, gstart):
    n = x.shape[0]
    score = jnp.tanh((x @ w) / (jnp.linalg.norm(w) + 1e-16))
    score_m = jnp.where(node_mask > 0, score, -2.0)
    comp = batch.astype(jnp.float32) * 10.0 - score_m
    ckey = _f32key(comp)
    ckey_p = jnp.pad(ckey, (0, N_PAD + 128 - n), constant_values=IMAX)
    thr = _topk_call()(ckey_p, ginfo)
    ks = thr[0, :, 0]
    mloc = thr[1, :, 0]
    lidx = jnp.arange(n, dtype=jnp.int32) - gstart[batch]
    keep = (ckey < ks[batch]) | ((ckey == ks[batch]) & (lidx <= mloc[batch]))
    keep = keep & (node_mask > 0)
    keep_f = keep.astype(jnp.float32)
    return x * score[:, None] * keep_f[:, None], keep_f


def _topk_pool(x, node_mask, batch, w):
    n = x.shape[0]
    score = jnp.tanh((x @ w) / (jnp.linalg.norm(w) + 1e-16))
    score_m = jnp.where(node_mask > 0, score, -2.0)
    comp = batch.astype(jnp.float32) * 10.0 - score_m
    order = jnp.argsort(comp)
    bs = batch[order]
    pos = jnp.arange(n)
    gstart = jax.ops.segment_min(pos, bs, num_segments=NUM_GRAPHS)
    rank_sorted = pos - gstart[bs]
    rank = jnp.zeros((n,), jnp.int32).at[order].set(rank_sorted.astype(jnp.int32))
    n_active = jax.ops.segment_sum(node_mask, batch, num_segments=NUM_GRAPHS)
    k = jnp.ceil(RATIO * n_active)
    keep = (rank.astype(jnp.float32) < k[batch]) & (node_mask > 0)
    keep_f = keep.astype(jnp.float32)
    x = x * score[:, None] * keep_f[:, None]
    return x, keep_f


def _pad2(a, r, c):
    return jnp.zeros((r, c), jnp.float32).at[: a.shape[0], : a.shape[1]].set(a)


def _head_kernel(h_ref, p_ref, *refs):
    out_ref = refs[-1]
    fcW = refs[0:5]
    fcb = refs[5:10]
    pmW = refs[10:15]
    pmb = refs[15:20]
    Wh = refs[20]
    Wp = refs[21]
    h = h_ref[:]
    for i in range(5):
        h = jax.nn.relu(
            jnp.dot(h, fcW[i][:], preferred_element_type=jnp.float32)
            + fcb[i][0:1, :])
    p = p_ref[:]
    for i in range(5):
        p = jax.nn.relu(
            jnp.dot(p, pmW[i][:], preferred_element_type=jnp.float32)
            + pmb[i][0:1, :])
    out_ref[:] = (jnp.dot(h, Wh[:], preferred_element_type=jnp.float32)
                  + jnp.dot(p, Wp[:], preferred_element_type=jnp.float32))


def _mlp_head(xg, prom_x, params):
    h = _pad2(xg, _PB, _PF)
    p = _pad2(prom_x, _PB, _PF)
    ops = [h, p]
    for l in params["fc"]:
        ops.append(_pad2(l["W"], _PF, _PF))
    for l in params["fc"]:
        ops.append(_pad2(l["b"][None, :], 8, _PF))
    for l in params["prom"]:
        ops.append(_pad2(l["W"], _PF, _PF))
    for l in params["prom"]:
        ops.append(_pad2(l["b"][None, :], 8, _PF))
    rW = params["readout_W"]
    ops.append(_pad2(rW[0:2], _PF, _PF))
    ops.append(_pad2(rW[2:4], _PF, _PF))
    out = pl.pallas_call(
        _head_kernel,
        out_shape=jax.ShapeDtypeStruct((_PB, _PF), jnp.float32),
    )(*ops)
    return out[:NUM_GRAPHS, 0:1] + params["readout_b"]


def kernel(x, edge_attr, edge_index, prom_x, batch, params):
    x = jnp.where(jnp.isnan(x), 0.0, x).astype(jnp.float32)
    edge_attr = jnp.where(jnp.isnan(edge_attr), 0.0, edge_attr).astype(jnp.float32)
    prom_x = jnp.reshape(prom_x, (-1, NCHIP)).astype(jnp.float32)
    prom_x = jnp.where(jnp.isnan(prom_x), 0.0, prom_x)
    src = edge_index[0]
    dst = edge_index[1]
    src_p = jnp.pad(src.astype(jnp.int32), (0, E_PAD - N_EDGES)).reshape(-1, 128)
    dst_p = jnp.pad(dst.astype(jnp.int32), (0, E_PAD - N_EDGES)).reshape(-1, 128)
    gs = jnp.searchsorted(batch, jnp.arange(NUM_GRAPHS)).astype(jnp.int32)
    ge = jnp.searchsorted(batch, jnp.arange(NUM_GRAPHS), side="right")
    gcnt = (ge - gs).astype(jnp.int32)
    kact = _f32key(10.0 * jnp.arange(NUM_GRAPHS, dtype=jnp.float32) + 1.5)
    ginfo = (jnp.zeros((G_PAD, 16), jnp.int32)
             .at[:NUM_GRAPHS, 0].set(gs)
             .at[NUM_GRAPHS:, 0].set(N_NODES)
             .at[:NUM_GRAPHS, 1].set(gcnt)
             .at[:NUM_GRAPHS, 2].set(kact))
    node_mask = jnp.ones((x.shape[0],), jnp.float32)
    for cp in params["convs"]:
        x, edge_attr = _conv_edge_sc(x, edge_attr, src_p, dst_p, node_mask, cp)
        x = jax.nn.relu(x)
        edge_attr = jax.nn.relu(edge_attr)
        x, node_mask = _topk_pool_sc(x, node_mask, batch, cp["pool_w"],
                                     ginfo, gs)
    xg = jax.ops.segment_max(
        jnp.where(node_mask[:, None] > 0, x, -1e9), batch,
        num_segments=NUM_GRAPHS)
    xg = jnp.where(jnp.isfinite(xg), xg, 0.0)
    return _mlp_head(xg, prom_x, params)
